# baseline (speedup 1.0000x reference)
import functools

import jax
import jax.numpy as jnp
from jax import lax
from jax.experimental import pallas as pl
from jax.experimental.pallas import tpu as pltpu
from jax.experimental.pallas import tpu_sc as plsc

_ETA = 10
_K = 32
_NC = 2
_NS = 16
_NW = _NC * _NS
_L = 16


@functools.lru_cache(maxsize=None)
def _build_sc_call(n: int):
    C = n // _NW
    G = C // _L
    mesh = plsc.VectorSubcoreMesh(core_axis_name="c", subcore_axis_name="s")

    @functools.partial(
        pl.kernel,
        out_type=(
            jax.ShapeDtypeStruct((n,), jnp.float32),
            jax.ShapeDtypeStruct((n * _ETA,), jnp.float32),
        ),
        mesh=mesh,
        compiler_params=pltpu.CompilerParams(
            needs_layout_passes=False, use_tc_tiling_on_sc=False),
        scratch_types=[
            pltpu.VMEM((2 * C,), jnp.int32),
            pltpu.VMEM((C,), jnp.int32),
            pltpu.VMEM((_ETA, C), jnp.int32),
            pltpu.VMEM((_ETA, C), jnp.int32),
            pltpu.VMEM((2 * C, _K), jnp.float32),
            pltpu.VMEM((C, _K), jnp.float32),
            pltpu.VMEM((C, _K), jnp.float32),
            pltpu.VMEM((C, _K), jnp.float32),
            pltpu.VMEM((C,), jnp.float32),
            pltpu.VMEM((C,), jnp.float32),
            pltpu.SemaphoreType.DMA,
            pltpu.SemaphoreType.DMA,
            pltpu.SemaphoreType.DMA,
            pltpu.SemaphoreType.DMA,
        ],
    )
    def call(ent_hbm, rel_hbm, so_hbm, p_hbm, repl_hbm, sel_hbm,
             out_inp, out_corr,
             so_v, p_v, repl_v, sel_v, eseo_v, ep_v, er0_v, er1_v,
             sinp_v, scor_v, sem_eseo, sem_ep, sem_er0, sem_er1):
        cid = lax.axis_index("c")
        sid = lax.axis_index("s")
        wid = sid * _NC + cid
        base_row = wid * C

        pltpu.sync_copy(so_hbm.at[wid], so_v)
        pltpu.sync_copy(p_hbm.at[wid], p_v)
        pltpu.sync_copy(repl_hbm.at[wid], repl_v)
        pltpu.sync_copy(sel_hbm.at[wid], sel_v)

        er_bufs = (er0_v, er1_v)
        er_sems = (sem_er0, sem_er1)
        cp_eseo = pltpu.async_copy(ent_hbm.at[so_v], eseo_v, sem_eseo)
        cp_ep = pltpu.async_copy(rel_hbm.at[p_v], ep_v, sem_ep)
        er_cps = [None] * _ETA
        er_cps[0] = pltpu.async_copy(ent_hbm.at[repl_v.at[0]], er0_v, sem_er0)
        cp_eseo.wait()
        cp_ep.wait()

        iot = lax.iota(jnp.int32, _L)

        def inp_group(g, carry):
            base = g * _L
            rows = base + iot
            acc = jnp.zeros((_L,), jnp.float32)
            for k in range(_K):
                kc = jnp.full((_L,), k, jnp.int32)
                sv = plsc.load_gather(eseo_v, [rows, kc])
                pv = plsc.load_gather(ep_v, [rows, kc])
                ov = plsc.load_gather(eseo_v, [rows + C, kc])
                acc = acc + sv * pv * ov
            sinp_v[pl.ds(base, _L)] = acc
            return carry

        lax.fori_loop(0, G, inp_group, 0)
        pltpu.sync_copy(sinp_v, out_inp.at[pl.ds(base_row, C)])

        for t in range(_ETA):
            er_cps[t].wait()
            if t + 1 < _ETA:
                er_cps[t + 1] = pltpu.async_copy(
                    ent_hbm.at[repl_v.at[t + 1]],
                    er_bufs[(t + 1) % 2], er_sems[(t + 1) % 2])
            er_v = er_bufs[t % 2]

            def corr_group(g, carry, t=t, er_v=er_v):
                base = g * _L
                rows = base + iot
                selr = sel_v[t, pl.ds(base, _L)]
                acc = jnp.zeros((_L,), jnp.float32)
                for k in range(_K):
                    kc = jnp.full((_L,), k, jnp.int32)
                    cv = plsc.load_gather(eseo_v, [selr, kc])
                    pv = plsc.load_gather(ep_v, [rows, kc])
                    rv = plsc.load_gather(er_v, [rows, kc])
                    acc = acc + cv * pv * rv
                scor_v[pl.ds(base, _L)] = acc
                return carry

            lax.fori_loop(0, G, corr_group, 0)
            pltpu.sync_copy(scor_v, out_corr.at[pl.ds(t * n + base_row, C)])

    return call


def kernel(inputs, ent_emb, rel_emb):
    n = inputs.shape[0]
    n_ent = ent_emb.shape[0]
    C = n // _NW

    km, kr = jax.random.split(jax.random.key(42))
    keep_subj = jax.random.randint(km, (n * _ETA,), 0, 2, dtype=jnp.int32)
    replacements = jax.random.randint(kr, (n * _ETA,), 0, n_ent,
                                      dtype=jnp.int32)
    keep_obj = 1 - keep_subj

    s = inputs[:, 0].reshape(_NW, C)
    p = inputs[:, 1].reshape(_NW, C)
    o = inputs[:, 2].reshape(_NW, C)
    so = jnp.concatenate([s, o], axis=1)
    repl = replacements.reshape(_ETA, _NW, C).transpose(1, 0, 2)
    sel = (jnp.arange(C, dtype=jnp.int32)[None, None, :]
           + C * keep_obj.reshape(_ETA, _NW, C).transpose(1, 0, 2))

    inp_score, corr_score = _build_sc_call(n)(
        ent_emb, rel_emb, so, p, repl, sel)
    return (inp_score, corr_score)
